# Initial kernel scaffold; baseline (speedup 1.0000x reference)
import jax
import jax.numpy as jnp
from jax.experimental import pallas as pl


def kernel(x, edge_index, batch, graph_stats, eps, params):
    raise NotImplementedError("write your pallas kernel here")


# speedup vs baseline: 286.6052x; 286.6052x over previous
import functools

import jax
import jax.numpy as jnp
from jax import lax
from jax.experimental import pallas as pl
from jax.experimental.pallas import tpu as pltpu
from jax.experimental.pallas import tpu_sc as plsc

N = 4096
E = 65536
B = 8
DIN = 128
H = 256
LAT = 64
C = 3
TL = 512
FD = 32
P = 7
NL = 3

BM = 256
NBLK = N // BM
NC, NS = 2, 16
CH = 128


def _leaky(x):
    return jnp.where(x > 0, x, 0.2 * x)


@functools.lru_cache(maxsize=None)
def _sc_call(f, mode):
    col = mode == 'col'
    ones = mode == 'ones'
    fh = f // 2 if col else f
    nw = NS if col else NC * NS
    epw = E // nw
    nch = epw // CH
    rpw = N // NS
    mesh = plsc.VectorSubcoreMesh(core_axis_name="c", subcore_axis_name="s",
                                  num_cores=NC, num_subcores=NS)

    tab_rows = 2 * N if col else N
    scratch = [
        pltpu.VMEM((CH,), jnp.int32),
        pltpu.VMEM((CH, fh), jnp.float32),
        pltpu.VMEM((rpw // 2, fh), jnp.float32),
        pltpu.VMEM_SHARED((N, fh), jnp.float32),
    ]
    if not ones:
        scratch = [pltpu.VMEM((CH,), jnp.int32)] + scratch
        scratch.append(pltpu.SemaphoreType.DMA)

    @functools.partial(
        pl.kernel,
        out_type=[jax.ShapeDtypeStruct((2 * N, fh), jnp.float32)],
        mesh=mesh,
        scratch_types=scratch,
    )
    def seg(*refs):
        it = iter(refs)
        if not ones:
            tab = next(it)
            src_hbm = next(it)
        dst_hbm = next(it)
        out = next(it)
        if not ones:
            src_v = next(it)
        dst_v, rows_v, stage_v, acc = next(it), next(it), next(it), next(it)
        sem = next(it) if not ones else None

        cid = lax.axis_index("c")
        sid = lax.axis_index("s")

        z16 = jnp.zeros((16,), jnp.float32)
        o16 = jnp.ones((16,), jnp.float32)

        def zrow(r, carry):
            for j in range(fh // 16):
                stage_v[r, pl.ds(j * 16, 16)] = z16
            return carry

        lax.fori_loop(0, rpw // 2, zrow, 0)
        if ones:
            def orow(r, carry):
                for j in range(fh // 16):
                    rows_v[r, pl.ds(j * 16, 16)] = o16
                return carry

            lax.fori_loop(0, CH, orow, 0)
        r0 = sid * rpw
        pltpu.sync_copy(stage_v, acc.at[pl.ds(r0, rpw // 2)])
        pltpu.sync_copy(stage_v, acc.at[pl.ds(r0 + rpw // 2, rpw // 2)])
        plsc.subcore_barrier()

        if col:
            e0 = sid * epw
        else:
            e0 = (sid * NC + cid) * epw
        ioff = cid * N

        def chunk(j, carry):
            base = pl.multiple_of(e0 + j * CH, CH)
            pltpu.sync_copy(dst_hbm.at[pl.ds(base, CH)], dst_v)
            if not ones:
                pltpu.sync_copy(src_hbm.at[pl.ds(base, CH)], src_v)
                if col:
                    for jj in range(CH // 16):
                        v = src_v[pl.ds(jj * 16, 16)]
                        src_v[pl.ds(jj * 16, 16)] = v + ioff
                pltpu.async_copy(tab.at[src_v], rows_v, sem).wait()
            pltpu.sync_copy(rows_v, acc.at[dst_v], add=True)
            return carry

        lax.fori_loop(0, nch, chunk, 0)
        plsc.subcore_barrier()

        for half in range(2):
            ro = r0 + half * (rpw // 2)
            pltpu.sync_copy(acc.at[pl.ds(ro, rpw // 2)], stage_v)
            pltpu.sync_copy(stage_v, out.at[pl.ds(ioff + ro, rpw // 2)])

    return seg


def _segsum2(table, src, dst):
    f = table.shape[1]
    if f % 256 == 0:
        fh = f // 2
        tab2 = jnp.concatenate([table[:, :fh], table[:, fh:]], axis=0)
        out = _sc_call(f, 'col')(tab2, src, dst)[0]
    else:
        out = _sc_call(f, 'edge')(table, src, dst)[0]
    return out[:N], out[N:]


def _degree2(dst):
    out = _sc_call(128, 'ones')(dst)[0]
    return out[:N], out[N:]


def _row_spec(d):
    return pl.BlockSpec((BM, d), lambda i: (i, 0))


def _full_spec(shape):
    nd = len(shape)
    return pl.BlockSpec(shape, lambda i: (0,) * nd)


def _gin_tc(h, aggA, aggB, cat, batchf, gsp, statWp, W1, W2, aux):
    fin = h.shape[1]

    def body(h_ref, aA_ref, aB_ref, b_ref, gs_ref, sw_ref, w1_ref, w2_ref,
             aux_ref, o_ref):
        if cat:
            agg = jnp.concatenate([aA_ref[...], aB_ref[...]], axis=1)
        else:
            agg = aA_ref[...] + aB_ref[...]
        t = h_ref[...] + agg
        a = _leaky(jnp.dot(t, w1_ref[...],
                           preferred_element_type=jnp.float32, precision=lax.Precision.HIGHEST) + aux_ref[0:1, :])
        a = a * aux_ref[2:3, :] + aux_ref[3:4, :]
        a = _leaky(jnp.dot(a, w2_ref[...],
                           preferred_element_type=jnp.float32, precision=lax.Precision.HIGHEST) + aux_ref[1:2, :])
        g = gs_ref[...]
        g = jnp.where(g != g, -100.0, g)
        sp8 = jnp.dot(g, sw_ref[...], preferred_element_type=jnp.float32, precision=lax.Precision.HIGHEST)
        lane = lax.broadcasted_iota(jnp.int32, (1, 128), 1).astype(jnp.float32)
        oh = (b_ref[...] == lane).astype(jnp.float32)
        sp = jnp.dot(oh, sp8, preferred_element_type=jnp.float32, precision=lax.Precision.HIGHEST)
        o_ref[...] = a + sp + aux_ref[4:5, :]

    return pl.pallas_call(
        body,
        grid=(NBLK,),
        in_specs=[
            _row_spec(fin), _row_spec(128), _row_spec(128), _row_spec(1),
            _full_spec((128, 128)), _full_spec((128, H)),
            _full_spec((fin, H)), _full_spec((H, H)), _full_spec((8, H)),
        ],
        out_specs=_row_spec(H),
        out_shape=jax.ShapeDtypeStruct((N, H), jnp.float32),
    )(h, aggA, aggB, batchf, gsp, statWp, W1, W2, aux)


def _head_tc(h, aggA, aggB, batchf, gsp, statWp, W1, W2, aux,
             eps, fcoW, muW, lvW, ld1W, ld2p, bm2p, aux64, aux128):
    def body(h_ref, aA_ref, aB_ref, b_ref, gs_ref, sw_ref, w1_ref, w2_ref,
             aux_ref, eps_ref, fco_ref, mu_ref, lv_ref, ld1_ref, ld2_ref,
             bm2_ref, a64_ref, a128_ref,
             mu_o, lv_o, z_o, y_o, yoh_o, brow_o, l_o):
        agg = jnp.concatenate([aA_ref[...], aB_ref[...]], axis=1)
        t = h_ref[...] + agg
        a = _leaky(jnp.dot(t, w1_ref[...],
                           preferred_element_type=jnp.float32, precision=lax.Precision.HIGHEST) + aux_ref[0:1, :])
        a = a * aux_ref[2:3, :] + aux_ref[3:4, :]
        a = _leaky(jnp.dot(a, w2_ref[...],
                           preferred_element_type=jnp.float32, precision=lax.Precision.HIGHEST) + aux_ref[1:2, :])
        g = gs_ref[...]
        g = jnp.where(g != g, -100.0, g)
        sp8 = jnp.dot(g, sw_ref[...], preferred_element_type=jnp.float32, precision=lax.Precision.HIGHEST)
        lane = lax.broadcasted_iota(jnp.int32, (1, 128), 1).astype(jnp.float32)
        oh = (b_ref[...] == lane).astype(jnp.float32)
        sp = jnp.dot(oh, sp8, preferred_element_type=jnp.float32, precision=lax.Precision.HIGHEST)
        h3 = a + sp + aux_ref[4:5, :]

        hc = jnp.dot(h3, fco_ref[...],
                     preferred_element_type=jnp.float32, precision=lax.Precision.HIGHEST) + aux_ref[5:6, :]
        mu = jnp.dot(hc, mu_ref[...],
                     preferred_element_type=jnp.float32, precision=lax.Precision.HIGHEST) + a64_ref[0:1, :]
        lv = jnp.dot(hc, lv_ref[...],
                     preferred_element_type=jnp.float32, precision=lax.Precision.HIGHEST) + a64_ref[1:2, :]
        z = mu + jnp.exp(0.5 * lv) * eps_ref[...]
        l1 = jnp.maximum(
            jnp.dot(z, ld1_ref[...],
                    preferred_element_type=jnp.float32, precision=lax.Precision.HIGHEST) + a64_ref[2:3, :], 0.0)
        logit = jnp.dot(l1, ld2_ref[...],
                        preferred_element_type=jnp.float32, precision=lax.Precision.HIGHEST) + a128_ref[0:1, :]
        mask3 = lane < float(C)
        lm = jnp.where(mask3, logit, -1e30)
        m = jnp.max(lm, axis=1, keepdims=True)
        ex = jnp.where(mask3, jnp.exp(lm - m), 0.0)
        s = jnp.sum(ex, axis=1, keepdims=True)
        y = ex / s

        def colv(j):
            sel = lane == float(j)
            return jnp.sum(jnp.where(sel, y, 0.0), axis=1, keepdims=True)

        y0, y1, y2 = colv(0), colv(1), colv(2)
        p0 = (y0 >= y1) & (y0 >= y2)
        p1 = jnp.logical_not(p0) & (y1 >= y2)
        p2 = jnp.logical_not(p0) & jnp.logical_not(p1)
        yoh = (p0.astype(jnp.float32) * (lane == 0.0).astype(jnp.float32)
               + p1.astype(jnp.float32) * (lane == 1.0).astype(jnp.float32)
               + p2.astype(jnp.float32) * (lane == 2.0).astype(jnp.float32))
        brow = jnp.dot(yoh, bm2_ref[...], preferred_element_type=jnp.float32, precision=lax.Precision.HIGHEST)

        mu_o[...] = mu
        lv_o[...] = lv
        z_o[...] = z
        y_o[...] = y
        yoh_o[...] = yoh
        brow_o[...] = brow
        l_o[...] = logit

    return pl.pallas_call(
        body,
        grid=(NBLK,),
        in_specs=[
            _row_spec(H), _row_spec(128), _row_spec(128), _row_spec(1),
            _full_spec((128, 128)), _full_spec((128, H)),
            _full_spec((H, H)), _full_spec((H, H)), _full_spec((8, H)),
            _row_spec(LAT), _full_spec((H, H)),
            _full_spec((H, LAT)), _full_spec((H, LAT)),
            _full_spec((LAT, LAT)), _full_spec((LAT, 128)),
            _full_spec((128, 128)), _full_spec((8, LAT)),
            _full_spec((8, 128)),
        ],
        out_specs=[
            _row_spec(LAT), _row_spec(LAT), _row_spec(LAT),
            _row_spec(128), _row_spec(128), _row_spec(128), _row_spec(128),
        ],
        out_shape=[
            jax.ShapeDtypeStruct((N, LAT), jnp.float32),
            jax.ShapeDtypeStruct((N, LAT), jnp.float32),
            jax.ShapeDtypeStruct((N, LAT), jnp.float32),
            jax.ShapeDtypeStruct((N, 128), jnp.float32),
            jax.ShapeDtypeStruct((N, 128), jnp.float32),
            jax.ShapeDtypeStruct((N, 128), jnp.float32),
            jax.ShapeDtypeStruct((N, 128), jnp.float32),
        ],
    )(h, aggA, aggB, batchf, gsp, statWp, W1, W2, aux,
      eps, fcoW, muW, lvW, ld1W, ld2p, bm2p, aux64, aux128)


def _dec_tc(z, ypad, degA, degB, st1a, st1bp, st2, fp1a, fp1bp, fp2, gcnW,
            aux64d, aux512):
    def body(z_ref, y_ref, dA_ref, dB_ref, s1a_ref, s1b_ref, s2_ref, f1a_ref,
             f1b_ref, f2_ref, gw_ref, a64_ref, a512_ref, zs_o, zf_o, xwp_o):
        z = z_ref[...]
        y = y_ref[...]
        t = jnp.maximum(
            jnp.dot(z, s1a_ref[...], preferred_element_type=jnp.float32, precision=lax.Precision.HIGHEST)
            + jnp.dot(y, s1b_ref[...], preferred_element_type=jnp.float32, precision=lax.Precision.HIGHEST)
            + a64_ref[0:1, :], 0.0)
        zs = jnp.dot(t, s2_ref[...],
                     preferred_element_type=jnp.float32, precision=lax.Precision.HIGHEST) + a64_ref[1:2, :]
        t2 = jnp.maximum(
            jnp.dot(z, f1a_ref[...], preferred_element_type=jnp.float32, precision=lax.Precision.HIGHEST)
            + jnp.dot(y, f1b_ref[...], preferred_element_type=jnp.float32, precision=lax.Precision.HIGHEST)
            + a64_ref[2:3, :], 0.0)
        zf = jnp.dot(t2, f2_ref[...],
                     preferred_element_type=jnp.float32, precision=lax.Precision.HIGHEST) + a512_ref[0:1, :]
        d = jnp.max(dA_ref[...] + dB_ref[...], axis=1, keepdims=True) + 1.0
        dis = lax.rsqrt(d)
        xwp = dis * jnp.dot(zf, gw_ref[...], preferred_element_type=jnp.float32, precision=lax.Precision.HIGHEST)
        zs_o[...] = zs
        zf_o[...] = zf
        xwp_o[...] = xwp

    return pl.pallas_call(
        body,
        grid=(NBLK,),
        in_specs=[
            _row_spec(LAT), _row_spec(128), _row_spec(128), _row_spec(128),
            _full_spec((LAT, LAT)), _full_spec((128, LAT)),
            _full_spec((LAT, LAT)), _full_spec((LAT, LAT)),
            _full_spec((128, LAT)), _full_spec((LAT, TL)),
            _full_spec((TL, TL)), _full_spec((8, LAT)), _full_spec((8, TL)),
        ],
        out_specs=[_row_spec(LAT), _row_spec(TL), _row_spec(TL)],
        out_shape=[
            jax.ShapeDtypeStruct((N, LAT), jnp.float32),
            jax.ShapeDtypeStruct((N, TL), jnp.float32),
            jax.ShapeDtypeStruct((N, TL), jnp.float32),
        ],
    )(z, ypad, degA, degB, st1a, st1bp, st2, fp1a, fp1bp, fp2, gcnW,
      aux64d, aux512)


def _adj_tc(zs, brow, yoh, batchf, batchc3):
    def body(zr_ref, zc_ref, br_ref, yc_ref, bfr_ref, bfc_ref, o_ref):
        i = pl.program_id(0)
        j = pl.program_id(1)
        s = lax.dot_general(zr_ref[...], zc_ref[...],
                            (((1,), (1,)), ((), ())),
                            preferred_element_type=jnp.float32, precision=lax.Precision.HIGHEST)
        a = jax.nn.sigmoid(s)
        bmv = lax.dot_general(br_ref[...], yc_ref[...],
                              (((1,), (1,)), ((), ())),
                              preferred_element_type=jnp.float32, precision=lax.Precision.HIGHEST)
        bc = bfc_ref[0]
        msk = (bfr_ref[...] == bc).astype(jnp.float32)
        gr = lax.broadcasted_iota(jnp.int32, (BM, 1), 0) + i * BM
        gc = lax.broadcasted_iota(jnp.int32, (1, BM), 1) + j * BM
        neq = (gr != gc).astype(jnp.float32)
        o_ref[...] = a * bmv * msk * neq

    return pl.pallas_call(
        body,
        grid=(NBLK, NBLK),
        in_specs=[
            pl.BlockSpec((BM, LAT), lambda i, j: (i, 0)),
            pl.BlockSpec((BM, LAT), lambda i, j: (j, 0)),
            pl.BlockSpec((BM, 128), lambda i, j: (i, 0)),
            pl.BlockSpec((BM, 128), lambda i, j: (j, 0)),
            pl.BlockSpec((BM, 1), lambda i, j: (i, 0)),
            pl.BlockSpec((1, 1, BM), lambda i, j: (j, 0, 0)),
        ],
        out_specs=pl.BlockSpec((BM, BM), lambda i, j: (i, j)),
        out_shape=jax.ShapeDtypeStruct((N, N), jnp.float32),
    )(zs, zs, brow, yoh, batchf, batchc3)


def _fin_tc(s0, s1, s2, s3, xwp, degA, degB, fbWp, aux512f, aux128f):
    def body(s0_ref, s1_ref, s2_ref, s3_ref, xw_ref, dA_ref, dB_ref, fbw_ref,
             a512_ref, a128_ref, o_ref):
        sm = jnp.concatenate([s0_ref[...], s1_ref[...], s2_ref[...],
                              s3_ref[...]], axis=1) + xw_ref[...]
        d = jnp.max(dA_ref[...] + dB_ref[...], axis=1, keepdims=True) + 1.0
        dis = lax.rsqrt(d)
        smn = dis * sm + a512_ref[0:1, :]
        zfr = jnp.maximum(smn, 0.0)
        o_ref[...] = jnp.dot(zfr, fbw_ref[...],
                             preferred_element_type=jnp.float32, precision=lax.Precision.HIGHEST) + a128_ref[0:1, :]

    return pl.pallas_call(
        body,
        grid=(NBLK,),
        in_specs=[
            _row_spec(128), _row_spec(128), _row_spec(128), _row_spec(128),
            _row_spec(TL), _row_spec(128), _row_spec(128),
            _full_spec((TL, 128)), _full_spec((8, TL)), _full_spec((8, 128)),
        ],
        out_specs=_row_spec(128),
        out_shape=jax.ShapeDtypeStruct((N, 128), jnp.float32),
    )(s0, s1, s2, s3, xwp, degA, degB, fbWp, aux512f, aux128f)


def kernel(x, edge_index, batch, graph_stats, eps, params):
    p = params
    src = edge_index[0].astype(jnp.int32)
    dst = edge_index[1].astype(jnp.int32)
    batchf = batch.astype(jnp.float32).reshape(N, 1)
    batchc3 = batchf.reshape(NBLK, 1, BM)
    gsp = jnp.zeros((128, 128), jnp.float32).at[:B, :P].set(graph_stats)

    auxs, statWps = [], []
    for i in range(NL):
        scale = p[f'gin{i}_g'] / jnp.sqrt(p[f'gin{i}_v'] + 1e-5)
        shift = p[f'gin{i}_be'] - p[f'gin{i}_m'] * scale
        rows = [p[f'gin{i}_b1'], p[f'gin{i}_b2'], scale, shift,
                p[f'stat{i}_b'], jnp.zeros(H, jnp.float32),
                jnp.zeros(H, jnp.float32), jnp.zeros(H, jnp.float32)]
        auxs.append(jnp.stack(rows))
        statWps.append(jnp.zeros((128, H), jnp.float32).at[:P].set(p[f'stat{i}_W']))
    auxs[2] = auxs[2].at[5].set(p['fco_b'])

    aux64 = jnp.zeros((8, LAT), jnp.float32)
    aux64 = aux64.at[0].set(p['mu_b']).at[1].set(p['lv_b']).at[2].set(p['ld1_b'])
    ld2p = jnp.zeros((LAT, 128), jnp.float32).at[:, :C].set(p['ld2_W'])
    aux128 = jnp.zeros((8, 128), jnp.float32).at[0, :C].set(p['ld2_b'])
    bmx = jax.nn.softmax(p['hb'], axis=1)
    bm2 = 0.5 * (bmx + bmx.T)
    bm2p = jnp.zeros((128, 128), jnp.float32).at[:C, :C].set(bm2)

    st1a = p['st1_W'][:LAT]
    st1bp = jnp.zeros((128, LAT), jnp.float32).at[:C].set(p['st1_W'][LAT:])
    fp1a = p['fp1_W'][:LAT]
    fp1bp = jnp.zeros((128, LAT), jnp.float32).at[:C].set(p['fp1_W'][LAT:])
    aux64d = jnp.zeros((8, LAT), jnp.float32)
    aux64d = aux64d.at[0].set(p['st1_b']).at[1].set(p['st2_b']).at[2].set(p['fp1_b'])
    aux512 = jnp.zeros((8, TL), jnp.float32).at[0].set(p['fp2_b'])
    aux512f = jnp.zeros((8, TL), jnp.float32).at[0].set(p['gcn_b'])
    fbWp = jnp.zeros((TL, 128), jnp.float32).at[:, :FD].set(p['fb_W'])
    aux128f = jnp.zeros((8, 128), jnp.float32).at[0, :FD].set(p['fb_b'])

    degA, degB = _degree2(dst)

    h = x
    for i in range(2):
        aA, aB = _segsum2(h, src, dst)
        h = _gin_tc(h, aA, aB, i > 0, batchf, gsp, statWps[i],
                    p[f'gin{i}_W1'], p[f'gin{i}_W2'], auxs[i])
    aA, aB = _segsum2(h, src, dst)
    mu, lv, z, ypad, yoh, brow, lpad = _head_tc(
        h, aA, aB, batchf, gsp, statWps[2], p['gin2_W1'], p['gin2_W2'],
        auxs[2], eps, p['fco_W'], p['mu_W'], p['lv_W'], p['ld1_W'], ld2p,
        bm2p, aux64, aux128)

    zs, zf, xwp = _dec_tc(z, ypad, degA, degB, st1a, st1bp, p['st2_W'],
                          fp1a, fp1bp, p['fp2_W'], p['gcn_W'], aux64d, aux512)
    adj = _adj_tc(zs, brow, yoh, batchf, batchc3)
    s0, s1 = _segsum2(xwp[:, :TL // 2], src, dst)
    s2, s3 = _segsum2(xwp[:, TL // 2:], src, dst)
    featsp = _fin_tc(s0, s1, s2, s3, xwp, degA, degB, fbWp, aux512f, aux128f)

    logits = lpad[:, :C]
    feats = featsp[:, :FD]
    return adj, logits, feats, mu, lv


# speedup vs baseline: 346.1791x; 1.2079x over previous
import functools

import jax
import jax.numpy as jnp
from jax import lax
from jax.experimental import pallas as pl
from jax.experimental.pallas import tpu as pltpu
from jax.experimental.pallas import tpu_sc as plsc

N = 4096
E = 65536
B = 8
DIN = 128
H = 256
LAT = 64
C = 3
TL = 512
FD = 32
P = 7
NL = 3

BM = 256
NBLK = N // BM
NC, NS = 2, 16
CH = 128


def _leaky(x):
    return jnp.where(x > 0, x, 0.2 * x)


@functools.lru_cache(maxsize=None)
def _sc_call(f, mode):
    col = mode == 'col'
    ones = mode == 'ones'
    fh = 128
    nsplit = f // fh if col else 2
    npass = (f // fh) // NC if col else 1
    nw = NS if col else NC * NS
    epw = E // nw
    nch = epw // CH
    rpw = N // NS
    mesh = plsc.VectorSubcoreMesh(core_axis_name="c", subcore_axis_name="s",
                                  num_cores=NC, num_subcores=NS)

    scratch = [
        pltpu.VMEM((nch, 1, CH), jnp.int32),
        pltpu.VMEM((CH, fh), jnp.float32),
        pltpu.VMEM((rpw // 2, fh), jnp.float32),
        pltpu.VMEM_SHARED((N, fh), jnp.float32),
    ]
    if not ones:
        scratch = ([pltpu.VMEM((nch, 1, CH), jnp.int32)] + scratch
                   + [pltpu.VMEM((CH, fh), jnp.float32),
                      pltpu.SemaphoreType.DMA, pltpu.SemaphoreType.DMA])

    @functools.partial(
        pl.kernel,
        out_type=[jax.ShapeDtypeStruct((nsplit * N, fh), jnp.float32)],
        mesh=mesh,
        scratch_types=scratch,
    )
    def seg(*refs):
        it = iter(refs)
        tab = next(it) if not ones else None
        if not ones:
            src2d = next(it)
        dst2d = next(it)
        out = next(it)
        if not ones:
            src_all = next(it)
        dst_all, rows0, stage_v, acc = next(it), next(it), next(it), next(it)
        if not ones:
            rows1, sem0, sem1 = next(it), next(it), next(it)

        cid = lax.axis_index("c")
        sid = lax.axis_index("s")

        if col:
            e0 = sid * epw
        else:
            e0 = (sid * NC + cid) * epw
        row0 = e0 // CH

        pltpu.sync_copy(dst2d.at[pl.ds(row0, nch)], dst_all)
        if not ones:
            pltpu.sync_copy(src2d.at[pl.ds(row0, nch)], src_all)

        def _bump_src(off):
            def body(r, carry):
                for jj in range(CH // 16):
                    v = src_all[r, 0, pl.ds(jj * 16, 16)]
                    src_all[r, 0, pl.ds(jj * 16, 16)] = v + off
                return carry

            lax.fori_loop(0, nch, body, 0)

        if col:
            _bump_src(cid * npass * N)

        z16 = jnp.zeros((16,), jnp.float32)
        o16 = jnp.ones((16,), jnp.float32)

        def zrow(r, carry):
            for j in range(fh // 16):
                stage_v[r, pl.ds(j * 16, 16)] = z16
            return carry

        lax.fori_loop(0, rpw // 2, zrow, 0)
        if ones:
            def orow(r, carry):
                for j in range(fh // 16):
                    rows0[r, pl.ds(j * 16, 16)] = o16
                return carry

            lax.fori_loop(0, CH, orow, 0)

        r0 = sid * rpw
        drain = tab.at[pl.ds(0, CH)] if not ones else None

        for q in range(npass):
            if q:
                _bump_src(N)
            pltpu.sync_copy(stage_v, acc.at[pl.ds(r0, rpw // 2)])
            pltpu.sync_copy(stage_v, acc.at[pl.ds(r0 + rpw // 2, rpw // 2)])
            plsc.subcore_barrier()

            if ones:
                def chunk(j, carry):
                    pltpu.sync_copy(rows0, acc.at[dst_all.at[j, 0]], add=True)
                    return carry

                lax.fori_loop(0, nch, chunk, 0)
            else:
                pltpu.async_copy(tab.at[src_all.at[0, 0]], rows0, sem0)

                def chunk(j, carry):
                    pltpu.async_copy(tab.at[src_all.at[2 * j + 1, 0]], rows1,
                                     sem1)
                    pltpu.make_async_copy(drain, rows0, sem0).wait()
                    pltpu.sync_copy(rows0, acc.at[dst_all.at[2 * j, 0]],
                                    add=True)
                    pltpu.async_copy(tab.at[src_all.at[2 * j + 2, 0]], rows0,
                                     sem0)
                    pltpu.make_async_copy(drain, rows1, sem1).wait()
                    pltpu.sync_copy(rows1, acc.at[dst_all.at[2 * j + 1, 0]],
                                    add=True)
                    return carry

                lax.fori_loop(0, nch // 2 - 1, chunk, 0)
                pltpu.async_copy(tab.at[src_all.at[nch - 1, 0]], rows1, sem1)
                pltpu.make_async_copy(drain, rows0, sem0).wait()
                pltpu.sync_copy(rows0, acc.at[dst_all.at[nch - 2, 0]], add=True)
                pltpu.make_async_copy(drain, rows1, sem1).wait()
                pltpu.sync_copy(rows1, acc.at[dst_all.at[nch - 1, 0]], add=True)

            plsc.subcore_barrier()
            ioff = (cid * npass + q) * N if col else cid * N
            wbuf = stage_v if ones else rows1
            for half in range(2):
                ro = r0 + half * (rpw // 2)
                pltpu.sync_copy(acc.at[pl.ds(ro, rpw // 2)], wbuf)
                pltpu.sync_copy(wbuf, out.at[pl.ds(ioff + ro, rpw // 2)])

    return seg


def _segsum2(table, src2d, dst2d):
    f = table.shape[1]
    if f % 256 == 0:
        nq = f // 128
        tab2 = jnp.concatenate(
            [table[:, i * 128:(i + 1) * 128] for i in range(nq)], axis=0)
        out = _sc_call(f, 'col')(tab2, src2d, dst2d)[0]
        return [out[i * N:(i + 1) * N] for i in range(nq)]
    out = _sc_call(f, 'edge')(table, src2d, dst2d)[0]
    return out[:N], out[N:]


def _degree2(dst2d):
    out = _sc_call(128, 'ones')(dst2d)[0]
    return out[:N], out[N:]


def _row_spec(d):
    return pl.BlockSpec((BM, d), lambda i: (i, 0))


def _full_spec(shape):
    nd = len(shape)
    return pl.BlockSpec(shape, lambda i: (0,) * nd)


def _gin_tc(h, aggA, aggB, cat, batchf, gsp, statWp, W1, W2, aux):
    fin = h.shape[1]

    def body(h_ref, aA_ref, aB_ref, b_ref, gs_ref, sw_ref, w1_ref, w2_ref,
             aux_ref, o_ref):
        if cat:
            agg = jnp.concatenate([aA_ref[...], aB_ref[...]], axis=1)
        else:
            agg = aA_ref[...] + aB_ref[...]
        t = h_ref[...] + agg
        a = _leaky(jnp.dot(t, w1_ref[...],
                           preferred_element_type=jnp.float32, precision=lax.Precision.HIGHEST) + aux_ref[0:1, :])
        a = a * aux_ref[2:3, :] + aux_ref[3:4, :]
        a = _leaky(jnp.dot(a, w2_ref[...],
                           preferred_element_type=jnp.float32, precision=lax.Precision.HIGHEST) + aux_ref[1:2, :])
        g = gs_ref[...]
        g = jnp.where(g != g, -100.0, g)
        sp8 = jnp.dot(g, sw_ref[...], preferred_element_type=jnp.float32, precision=lax.Precision.HIGHEST)
        lane = lax.broadcasted_iota(jnp.int32, (1, 128), 1).astype(jnp.float32)
        oh = (b_ref[...] == lane).astype(jnp.float32)
        sp = jnp.dot(oh, sp8, preferred_element_type=jnp.float32, precision=lax.Precision.HIGHEST)
        o_ref[...] = a + sp + aux_ref[4:5, :]

    return pl.pallas_call(
        body,
        grid=(NBLK,),
        in_specs=[
            _row_spec(fin), _row_spec(128), _row_spec(128), _row_spec(1),
            _full_spec((128, 128)), _full_spec((128, H)),
            _full_spec((fin, H)), _full_spec((H, H)), _full_spec((8, H)),
        ],
        out_specs=_row_spec(H),
        out_shape=jax.ShapeDtypeStruct((N, H), jnp.float32),
    )(h, aggA, aggB, batchf, gsp, statWp, W1, W2, aux)


def _head_tc(h, aggA, aggB, batchf, gsp, statWp, W1, W2, aux,
             eps, fcoW, muW, lvW, ld1W, ld2p, bm2p, aux64, aux128):
    def body(h_ref, aA_ref, aB_ref, b_ref, gs_ref, sw_ref, w1_ref, w2_ref,
             aux_ref, eps_ref, fco_ref, mu_ref, lv_ref, ld1_ref, ld2_ref,
             bm2_ref, a64_ref, a128_ref,
             mu_o, lv_o, z_o, y_o, yoh_o, brow_o, l_o):
        agg = jnp.concatenate([aA_ref[...], aB_ref[...]], axis=1)
        t = h_ref[...] + agg
        a = _leaky(jnp.dot(t, w1_ref[...],
                           preferred_element_type=jnp.float32, precision=lax.Precision.HIGHEST) + aux_ref[0:1, :])
        a = a * aux_ref[2:3, :] + aux_ref[3:4, :]
        a = _leaky(jnp.dot(a, w2_ref[...],
                           preferred_element_type=jnp.float32, precision=lax.Precision.HIGHEST) + aux_ref[1:2, :])
        g = gs_ref[...]
        g = jnp.where(g != g, -100.0, g)
        sp8 = jnp.dot(g, sw_ref[...], preferred_element_type=jnp.float32, precision=lax.Precision.HIGHEST)
        lane = lax.broadcasted_iota(jnp.int32, (1, 128), 1).astype(jnp.float32)
        oh = (b_ref[...] == lane).astype(jnp.float32)
        sp = jnp.dot(oh, sp8, preferred_element_type=jnp.float32, precision=lax.Precision.HIGHEST)
        h3 = a + sp + aux_ref[4:5, :]

        hc = jnp.dot(h3, fco_ref[...],
                     preferred_element_type=jnp.float32, precision=lax.Precision.HIGHEST) + aux_ref[5:6, :]
        mu = jnp.dot(hc, mu_ref[...],
                     preferred_element_type=jnp.float32, precision=lax.Precision.HIGHEST) + a64_ref[0:1, :]
        lv = jnp.dot(hc, lv_ref[...],
                     preferred_element_type=jnp.float32, precision=lax.Precision.HIGHEST) + a64_ref[1:2, :]
        z = mu + jnp.exp(0.5 * lv) * eps_ref[...]
        l1 = jnp.maximum(
            jnp.dot(z, ld1_ref[...],
                    preferred_element_type=jnp.float32, precision=lax.Precision.HIGHEST) + a64_ref[2:3, :], 0.0)
        logit = jnp.dot(l1, ld2_ref[...],
                        preferred_element_type=jnp.float32, precision=lax.Precision.HIGHEST) + a128_ref[0:1, :]
        mask3 = lane < float(C)
        lm = jnp.where(mask3, logit, -1e30)
        m = jnp.max(lm, axis=1, keepdims=True)
        ex = jnp.where(mask3, jnp.exp(lm - m), 0.0)
        s = jnp.sum(ex, axis=1, keepdims=True)
        y = ex / s

        def colv(j):
            sel = lane == float(j)
            return jnp.sum(jnp.where(sel, y, 0.0), axis=1, keepdims=True)

        y0, y1, y2 = colv(0), colv(1), colv(2)
        p0 = (y0 >= y1) & (y0 >= y2)
        p1 = jnp.logical_not(p0) & (y1 >= y2)
        p2 = jnp.logical_not(p0) & jnp.logical_not(p1)
        yoh = (p0.astype(jnp.float32) * (lane == 0.0).astype(jnp.float32)
               + p1.astype(jnp.float32) * (lane == 1.0).astype(jnp.float32)
               + p2.astype(jnp.float32) * (lane == 2.0).astype(jnp.float32))
        brow = jnp.dot(yoh, bm2_ref[...], preferred_element_type=jnp.float32, precision=lax.Precision.HIGHEST)

        mu_o[...] = mu
        lv_o[...] = lv
        z_o[...] = z
        y_o[...] = y
        yoh_o[...] = yoh
        brow_o[...] = brow
        l_o[...] = logit

    return pl.pallas_call(
        body,
        grid=(NBLK,),
        in_specs=[
            _row_spec(H), _row_spec(128), _row_spec(128), _row_spec(1),
            _full_spec((128, 128)), _full_spec((128, H)),
            _full_spec((H, H)), _full_spec((H, H)), _full_spec((8, H)),
            _row_spec(LAT), _full_spec((H, H)),
            _full_spec((H, LAT)), _full_spec((H, LAT)),
            _full_spec((LAT, LAT)), _full_spec((LAT, 128)),
            _full_spec((128, 128)), _full_spec((8, LAT)),
            _full_spec((8, 128)),
        ],
        out_specs=[
            _row_spec(LAT), _row_spec(LAT), _row_spec(LAT),
            _row_spec(128), _row_spec(128), _row_spec(128), _row_spec(128),
        ],
        out_shape=[
            jax.ShapeDtypeStruct((N, LAT), jnp.float32),
            jax.ShapeDtypeStruct((N, LAT), jnp.float32),
            jax.ShapeDtypeStruct((N, LAT), jnp.float32),
            jax.ShapeDtypeStruct((N, 128), jnp.float32),
            jax.ShapeDtypeStruct((N, 128), jnp.float32),
            jax.ShapeDtypeStruct((N, 128), jnp.float32),
            jax.ShapeDtypeStruct((N, 128), jnp.float32),
        ],
    )(h, aggA, aggB, batchf, gsp, statWp, W1, W2, aux,
      eps, fcoW, muW, lvW, ld1W, ld2p, bm2p, aux64, aux128)


def _dec_tc(z, ypad, degA, degB, st1a, st1bp, st2, fp1a, fp1bp, fp2, gcnW,
            aux64d, aux512):
    def body(z_ref, y_ref, dA_ref, dB_ref, s1a_ref, s1b_ref, s2_ref, f1a_ref,
             f1b_ref, f2_ref, gw_ref, a64_ref, a512_ref, zs_o, zf_o, xwp_o):
        z = z_ref[...]
        y = y_ref[...]
        t = jnp.maximum(
            jnp.dot(z, s1a_ref[...], preferred_element_type=jnp.float32, precision=lax.Precision.HIGHEST)
            + jnp.dot(y, s1b_ref[...], preferred_element_type=jnp.float32, precision=lax.Precision.HIGHEST)
            + a64_ref[0:1, :], 0.0)
        zs = jnp.dot(t, s2_ref[...],
                     preferred_element_type=jnp.float32, precision=lax.Precision.HIGHEST) + a64_ref[1:2, :]
        t2 = jnp.maximum(
            jnp.dot(z, f1a_ref[...], preferred_element_type=jnp.float32, precision=lax.Precision.HIGHEST)
            + jnp.dot(y, f1b_ref[...], preferred_element_type=jnp.float32, precision=lax.Precision.HIGHEST)
            + a64_ref[2:3, :], 0.0)
        zf = jnp.dot(t2, f2_ref[...],
                     preferred_element_type=jnp.float32, precision=lax.Precision.HIGHEST) + a512_ref[0:1, :]
        d = jnp.max(dA_ref[...] + dB_ref[...], axis=1, keepdims=True) + 1.0
        dis = lax.rsqrt(d)
        xwp = dis * jnp.dot(zf, gw_ref[...], preferred_element_type=jnp.float32, precision=lax.Precision.HIGHEST)
        zs_o[...] = zs
        zf_o[...] = zf
        xwp_o[...] = xwp

    return pl.pallas_call(
        body,
        grid=(NBLK,),
        in_specs=[
            _row_spec(LAT), _row_spec(128), _row_spec(128), _row_spec(128),
            _full_spec((LAT, LAT)), _full_spec((128, LAT)),
            _full_spec((LAT, LAT)), _full_spec((LAT, LAT)),
            _full_spec((128, LAT)), _full_spec((LAT, TL)),
            _full_spec((TL, TL)), _full_spec((8, LAT)), _full_spec((8, TL)),
        ],
        out_specs=[_row_spec(LAT), _row_spec(TL), _row_spec(TL)],
        out_shape=[
            jax.ShapeDtypeStruct((N, LAT), jnp.float32),
            jax.ShapeDtypeStruct((N, TL), jnp.float32),
            jax.ShapeDtypeStruct((N, TL), jnp.float32),
        ],
    )(z, ypad, degA, degB, st1a, st1bp, st2, fp1a, fp1bp, fp2, gcnW,
      aux64d, aux512)


def _adj_tc(zs, brow, yoh, batchf, batchc3):
    def body(zr_ref, zc_ref, br_ref, yc_ref, bfr_ref, bfc_ref, o_ref):
        i = pl.program_id(0)
        j = pl.program_id(1)
        s = lax.dot_general(zr_ref[...], zc_ref[...],
                            (((1,), (1,)), ((), ())),
                            preferred_element_type=jnp.float32, precision=lax.Precision.HIGHEST)
        a = jax.nn.sigmoid(s)
        bmv = lax.dot_general(br_ref[...], yc_ref[...],
                              (((1,), (1,)), ((), ())),
                              preferred_element_type=jnp.float32, precision=lax.Precision.HIGHEST)
        bc = bfc_ref[0]
        msk = (bfr_ref[...] == bc).astype(jnp.float32)
        gr = lax.broadcasted_iota(jnp.int32, (BM, 1), 0) + i * BM
        gc = lax.broadcasted_iota(jnp.int32, (1, BM), 1) + j * BM
        neq = (gr != gc).astype(jnp.float32)
        o_ref[...] = a * bmv * msk * neq

    return pl.pallas_call(
        body,
        grid=(NBLK, NBLK),
        in_specs=[
            pl.BlockSpec((BM, LAT), lambda i, j: (i, 0)),
            pl.BlockSpec((BM, LAT), lambda i, j: (j, 0)),
            pl.BlockSpec((BM, 128), lambda i, j: (i, 0)),
            pl.BlockSpec((BM, 128), lambda i, j: (j, 0)),
            pl.BlockSpec((BM, 1), lambda i, j: (i, 0)),
            pl.BlockSpec((1, 1, BM), lambda i, j: (j, 0, 0)),
        ],
        out_specs=pl.BlockSpec((BM, BM), lambda i, j: (i, j)),
        out_shape=jax.ShapeDtypeStruct((N, N), jnp.float32),
    )(zs, zs, brow, yoh, batchf, batchc3)


def _fin_tc(s0, s1, s2, s3, xwp, degA, degB, fbWp, aux512f, aux128f):
    def body(s0_ref, s1_ref, s2_ref, s3_ref, xw_ref, dA_ref, dB_ref, fbw_ref,
             a512_ref, a128_ref, o_ref):
        sm = jnp.concatenate([s0_ref[...], s1_ref[...], s2_ref[...],
                              s3_ref[...]], axis=1) + xw_ref[...]
        d = jnp.max(dA_ref[...] + dB_ref[...], axis=1, keepdims=True) + 1.0
        dis = lax.rsqrt(d)
        smn = dis * sm + a512_ref[0:1, :]
        zfr = jnp.maximum(smn, 0.0)
        o_ref[...] = jnp.dot(zfr, fbw_ref[...],
                             preferred_element_type=jnp.float32, precision=lax.Precision.HIGHEST) + a128_ref[0:1, :]

    return pl.pallas_call(
        body,
        grid=(NBLK,),
        in_specs=[
            _row_spec(128), _row_spec(128), _row_spec(128), _row_spec(128),
            _row_spec(TL), _row_spec(128), _row_spec(128),
            _full_spec((TL, 128)), _full_spec((8, TL)), _full_spec((8, 128)),
        ],
        out_specs=_row_spec(128),
        out_shape=jax.ShapeDtypeStruct((N, 128), jnp.float32),
    )(s0, s1, s2, s3, xwp, degA, degB, fbWp, aux512f, aux128f)


def kernel(x, edge_index, batch, graph_stats, eps, params):
    p = params
    src = edge_index[0].astype(jnp.int32)
    dst = edge_index[1].astype(jnp.int32)
    batchf = batch.astype(jnp.float32).reshape(N, 1)
    batchc3 = batchf.reshape(NBLK, 1, BM)
    gsp = jnp.zeros((128, 128), jnp.float32).at[:B, :P].set(graph_stats)

    auxs, statWps = [], []
    for i in range(NL):
        scale = p[f'gin{i}_g'] / jnp.sqrt(p[f'gin{i}_v'] + 1e-5)
        shift = p[f'gin{i}_be'] - p[f'gin{i}_m'] * scale
        rows = [p[f'gin{i}_b1'], p[f'gin{i}_b2'], scale, shift,
                p[f'stat{i}_b'], jnp.zeros(H, jnp.float32),
                jnp.zeros(H, jnp.float32), jnp.zeros(H, jnp.float32)]
        auxs.append(jnp.stack(rows))
        statWps.append(jnp.zeros((128, H), jnp.float32).at[:P].set(p[f'stat{i}_W']))
    auxs[2] = auxs[2].at[5].set(p['fco_b'])

    aux64 = jnp.zeros((8, LAT), jnp.float32)
    aux64 = aux64.at[0].set(p['mu_b']).at[1].set(p['lv_b']).at[2].set(p['ld1_b'])
    ld2p = jnp.zeros((LAT, 128), jnp.float32).at[:, :C].set(p['ld2_W'])
    aux128 = jnp.zeros((8, 128), jnp.float32).at[0, :C].set(p['ld2_b'])
    bmx = jax.nn.softmax(p['hb'], axis=1)
    bm2 = 0.5 * (bmx + bmx.T)
    bm2p = jnp.zeros((128, 128), jnp.float32).at[:C, :C].set(bm2)

    st1a = p['st1_W'][:LAT]
    st1bp = jnp.zeros((128, LAT), jnp.float32).at[:C].set(p['st1_W'][LAT:])
    fp1a = p['fp1_W'][:LAT]
    fp1bp = jnp.zeros((128, LAT), jnp.float32).at[:C].set(p['fp1_W'][LAT:])
    aux64d = jnp.zeros((8, LAT), jnp.float32)
    aux64d = aux64d.at[0].set(p['st1_b']).at[1].set(p['st2_b']).at[2].set(p['fp1_b'])
    aux512 = jnp.zeros((8, TL), jnp.float32).at[0].set(p['fp2_b'])
    aux512f = jnp.zeros((8, TL), jnp.float32).at[0].set(p['gcn_b'])
    fbWp = jnp.zeros((TL, 128), jnp.float32).at[:, :FD].set(p['fb_W'])
    aux128f = jnp.zeros((8, 128), jnp.float32).at[0, :FD].set(p['fb_b'])

    src2d = src.reshape(E // CH, 1, CH)
    dst2d = dst.reshape(E // CH, 1, CH)
    degA, degB = _degree2(dst2d)

    h = x
    for i in range(2):
        aA, aB = _segsum2(h, src2d, dst2d)
        h = _gin_tc(h, aA, aB, i > 0, batchf, gsp, statWps[i],
                    p[f'gin{i}_W1'], p[f'gin{i}_W2'], auxs[i])
    aA, aB = _segsum2(h, src2d, dst2d)
    mu, lv, z, ypad, yoh, brow, lpad = _head_tc(
        h, aA, aB, batchf, gsp, statWps[2], p['gin2_W1'], p['gin2_W2'],
        auxs[2], eps, p['fco_W'], p['mu_W'], p['lv_W'], p['ld1_W'], ld2p,
        bm2p, aux64, aux128)

    zs, zf, xwp = _dec_tc(z, ypad, degA, degB, st1a, st1bp, p['st2_W'],
                          fp1a, fp1bp, p['fp2_W'], p['gcn_W'], aux64d, aux512)
    adj = _adj_tc(zs, brow, yoh, batchf, batchc3)
    s0, s1, s2, s3 = _segsum2(xwp, src2d, dst2d)
    featsp = _fin_tc(s0, s1, s2, s3, xwp, degA, degB, fbWp, aux512f, aux128f)

    logits = lpad[:, :C]
    feats = featsp[:, :FD]
    return adj, logits, feats, mu, lv
